# baseline (speedup 1.0000x reference)
import functools

import jax
import jax.numpy as jnp
from jax import lax
from jax.experimental import pallas as pl
from jax.experimental.pallas import tpu as pltpu
from jax.experimental.pallas import tpu_sc as plsc

NC, NS, L = 2, 16, 16
NW = NC * NS

_LN1P_COEF = (1.0, -0.4999999, 0.33333334, -0.2500127, 0.20001155,
              -0.16616526, 0.14202611, -0.13198958, 0.12748663, -0.07539792)
_LN2 = 0.6931471805599453


def _neg_ln(u):
    bits = lax.bitcast_convert_type(u, jnp.int32)
    ix = bits + jnp.int32(0x3F800000 - 0x3F3504F3)
    k = lax.shift_right_arithmetic(ix, jnp.int32(23)) - jnp.int32(0x7F)
    mbits = (ix & jnp.int32(0x007FFFFF)) + jnp.int32(0x3F3504F3)
    m = lax.bitcast_convert_type(mbits, jnp.float32)
    w = m - jnp.float32(1.0)
    acc = jnp.full_like(w, _LN1P_COEF[-1])
    for c in _LN1P_COEF[-2::-1]:
        acc = acc * w + jnp.float32(c)
    ln1p = w * acc
    kf = k.astype(jnp.float32)
    return -(kf * jnp.float32(_LN2) + ln1p)


def make_sc_shard(V, B, v0, v_sc):
    chunk = v_sc // NW
    assert chunk * NW == v_sc and chunk % 8 == 0
    mesh = plsc.VectorSubcoreMesh(core_axis_name="c", subcore_axis_name="s", num_cores=NC)

    @functools.partial(
        pl.kernel,
        out_type=(jax.ShapeDtypeStruct((NW, B), jnp.float32),
                  jax.ShapeDtypeStruct((NW, B), jnp.int32)),
        mesh=mesh,
        scratch_types=[
            pltpu.VMEM((chunk, B), jnp.float32),
            pltpu.VMEM((chunk, B), jnp.float32),
            pltpu.VMEM((B,), jnp.float32),
            pltpu.VMEM((B,), jnp.int32),
        ],
    )
    def sc_kernel(l_hbm, u_hbm, outr_hbm, outi_hbm, l_v, u_v, outv, outi):
        wid = lax.axis_index("s") * NC + lax.axis_index("c")
        base = v0 + wid * chunk
        pltpu.sync_copy(l_hbm.at[pl.ds(base, chunk)], l_v)
        pltpu.sync_copy(u_hbm.at[pl.ds(base, chunk)], u_v)

        ngrp = B // L
        init = tuple(jnp.full((L,), jnp.inf, jnp.float32) for _ in range(ngrp)) \
             + tuple(jnp.zeros((L,), jnp.int32) for _ in range(ngrp))

        def step(v, carry):
            rmins = carry[:ngrp]
            ridxs = carry[ngrp:]
            idx = jnp.full((L,), v + base, jnp.int32)
            new_r, new_i = [], []
            for g in range(ngrp):
                lv = l_v[v, pl.ds(g * L, L)]
                uv = u_v[v, pl.ds(g * L, L)]
                r = _neg_ln(uv) * jnp.exp(-lv)
                upd = r < rmins[g]
                new_r.append(jnp.where(upd, r, rmins[g]))
                new_i.append(jnp.where(upd, idx, ridxs[g]))
            return tuple(new_r) + tuple(new_i)

        carry = lax.fori_loop(0, chunk, step, init)
        for g in range(ngrp):
            outv[pl.ds(g * L, L)] = carry[g]
            outi[pl.ds(g * L, L)] = carry[ngrp + g]
        pltpu.sync_copy(outv, outr_hbm.at[wid])
        pltpu.sync_copy(outi, outi_hbm.at[wid])

    return sc_kernel


def make_tc_main(V_tc, B, VB, SH):
    nb = pl.cdiv(V_tc, VB)
    tail = V_tc - (nb - 1) * VB
    assert tail % SH == 0

    def body(l_ref, u_ref, ov_ref, oi_ref, rmax, ridx):
        i = pl.program_id(0)

        @pl.when(i == 0)
        def _init():
            rmax[:] = jnp.full_like(rmax[:], -jnp.inf)
            ridx[:] = jnp.zeros_like(ridx[:])

        iotaS = jax.lax.broadcasted_iota(jnp.int32, (SH, B), 0)

        def scan_strips(n_strips):
            cm = rmax[:]
            ci = ridx[:]
            base = i * VB
            for k in range(n_strips):
                off = k * SH
                s = l_ref[off:off + SH, :] - jnp.log(-jnp.log(u_ref[off:off + SH, :]))
                idx = iotaS + (base + off)
                upd = s > cm
                cm = jnp.where(upd, s, cm)
                ci = jnp.where(upd, idx, ci)
            rmax[:] = cm
            ridx[:] = ci

        @pl.when(i < nb - 1)
        def _full():
            scan_strips(VB // SH)

        @pl.when(i == nb - 1)
        def _last():
            scan_strips(tail // SH)
            w = SH
            while w > 1:
                h = w // 2
                av, bv = rmax[0:h, :], rmax[h:w, :]
                ai, bi = ridx[0:h, :], ridx[h:w, :]
                take_b = (bv > av) | ((bv == av) & (bi < ai))
                rmax[0:h, :] = jnp.where(take_b, bv, av)
                ridx[0:h, :] = jnp.where(take_b, bi, ai)
                w = h
            ov_ref[:] = rmax[0:1, :]
            oi_ref[:] = ridx[0:1, :]

    return pl.pallas_call(
        body,
        grid=(nb,),
        in_specs=[
            pl.BlockSpec((VB, B), lambda i: (i, 0)),
            pl.BlockSpec((VB, B), lambda i: (i, 0)),
        ],
        out_specs=[
            pl.BlockSpec((1, B), lambda i: (0, 0)),
            pl.BlockSpec((1, B), lambda i: (0, 0)),
        ],
        out_shape=[
            jax.ShapeDtypeStruct((1, B), jnp.float32),
            jax.ShapeDtypeStruct((1, B), jnp.int32),
        ],
        scratch_shapes=[
            pltpu.VMEM((SH, B), jnp.float32),
            pltpu.VMEM((SH, B), jnp.int32),
        ],
        compiler_params=pltpu.CompilerParams(
            dimension_semantics=("arbitrary",),
        ),
    )


def make_merge(V, B):
    def body(tv_ref, ti_ref, r_ref, ri_ref, o_ref):
        r = r_ref[:]
        rmin = jnp.min(r, axis=0, keepdims=True)
        imin = jnp.min(jnp.where(r == rmin, ri_ref[:], V), axis=0, keepdims=True)
        s_sc = -jnp.log(rmin)
        tv = tv_ref[:]
        ti = ti_ref[:]
        take_sc = (s_sc > tv) | ((s_sc == tv) & (imin < ti))
        o_ref[:] = jnp.where(take_sc, imin, ti)

    return pl.pallas_call(
        body,
        out_shape=jax.ShapeDtypeStruct((1, B), jnp.int32),
    )


def make_kernel(V_SC=10240, VB=8192, SH=16):
    def kernel(log_conditionals, u):
        B, V = log_conditionals.shape
        lt, ut = log_conditionals.T, u.T
        v0 = V - V_SC
        sc_r, sc_i = make_sc_shard(V, B, v0, V_SC)(lt, ut)
        tc_v, tc_i = make_tc_main(v0, B, VB, SH)(lt, ut)
        out = make_merge(V, B)(tc_v, tc_i, sc_r, sc_i)
        return out.reshape(B)
    return kernel


kernel = make_kernel()


# speedup vs baseline: 1.4651x; 1.4651x over previous
import jax
import jax.numpy as jnp
from jax.experimental import pallas as pl
from jax.experimental.pallas import tpu as pltpu


def kernel(log_conditionals, u):
    B, V = log_conditionals.shape
    VB = 16384
    SH = 16
    nb = pl.cdiv(V, VB)
    tail = V - (nb - 1) * VB
    assert tail % SH == 0

    def body(l_ref, u_ref, o_ref, rmax, ridx):
        i = pl.program_id(0)

        @pl.when(i == 0)
        def _init():
            rmax[:] = jnp.full_like(rmax[:], -jnp.inf)
            ridx[:] = jnp.zeros_like(ridx[:])

        iota8 = jax.lax.broadcasted_iota(jnp.int32, (SH, B), 0)

        def scan_strips(n_strips):
            cm = rmax[:]
            ci = ridx[:]
            base = i * VB
            for k in range(n_strips):
                off = k * SH
                s = l_ref[off:off + SH, :] - jnp.log(-jnp.log(u_ref[off:off + SH, :]))
                idx = iota8 + (base + off)
                upd = s > cm
                cm = jnp.where(upd, s, cm)
                ci = jnp.where(upd, idx, ci)
            rmax[:] = cm
            ridx[:] = ci

        @pl.when(i < nb - 1)
        def _full():
            scan_strips(VB // SH)

        @pl.when(i == nb - 1)
        def _last():
            scan_strips(tail // SH)
            w = SH
            while w > 1:
                h = w // 2
                av, bv = rmax[0:h, :], rmax[h:w, :]
                ai, bi = ridx[0:h, :], ridx[h:w, :]
                take_b = (bv > av) | ((bv == av) & (bi < ai))
                rmax[0:h, :] = jnp.where(take_b, bv, av)
                ridx[0:h, :] = jnp.where(take_b, bi, ai)
                w = h
            o_ref[:] = ridx[0:1, :]

    out = pl.pallas_call(
        body,
        grid=(nb,),
        in_specs=[
            pl.BlockSpec((VB, B), lambda i: (i, 0)),
            pl.BlockSpec((VB, B), lambda i: (i, 0)),
        ],
        out_specs=pl.BlockSpec((1, B), lambda i: (0, 0)),
        out_shape=jax.ShapeDtypeStruct((1, B), jnp.int32),
        scratch_shapes=[
            pltpu.VMEM((SH, B), jnp.float32),
            pltpu.VMEM((SH, B), jnp.int32),
        ],
        compiler_params=pltpu.CompilerParams(
            dimension_semantics=("arbitrary",),
        ),
    )(log_conditionals.T, u.T)
    return out.reshape(B)
